# Initial kernel scaffold; baseline (speedup 1.0000x reference)
import jax
import jax.numpy as jnp
from jax.experimental import pallas as pl


def kernel(context, target, neg_samples, W, C):
    raise NotImplementedError("write your pallas kernel here")


# speedup vs baseline: 5.0530x; 5.0530x over previous
import functools

import jax
import jax.numpy as jnp
from jax import lax
from jax.experimental import pallas as pl
from jax.experimental.pallas import tpu as pltpu
from jax.experimental.pallas import tpu_sc as plsc

_NC = 2
_NS = 16
_L = 16


def _gather_rows(table_hbm, idx_ref, rows_ref, sem, total):
    copies = []
    off = 0
    while off < total:
        cnt = min(128, total - off)
        copies.append(
            pltpu.async_copy(
                table_hbm.at[idx_ref.at[pl.ds(off, cnt)]],
                rows_ref.at[pl.ds(off, cnt)],
                sem,
            )
        )
        off += cnt
    for c in copies:
        c.wait()


@functools.partial(jax.jit, static_argnums=(4, 5, 6))
def _cbow_logits(ctx_idx, ci, W, C, B, CTX, NSCORE):
    D = W.shape[1]
    NW = _NC * _NS
    RW = B // NW
    NB = 16
    NCHUNK = RW // NB
    NG = D // _L
    OW = 2 * _L

    mesh = plsc.VectorSubcoreMesh(
        core_axis_name="c", subcore_axis_name="s",
        num_cores=_NC, num_subcores=_NS,
    )

    @functools.partial(
        pl.kernel,
        out_type=jax.ShapeDtypeStruct((B, OW), jnp.float32),
        mesh=mesh,
        compiler_params=pltpu.CompilerParams(use_tc_tiling_on_sc=False),
        scratch_types=[
            pltpu.VMEM((NB * CTX,), jnp.int32),
            pltpu.VMEM((NB * NSCORE,), jnp.int32),
            pltpu.VMEM((NB * CTX, D), jnp.float32),
            pltpu.VMEM((NB * NSCORE, D), jnp.float32),
            pltpu.VMEM((NB, OW), jnp.float32),
            pltpu.SemaphoreType.DMA,
        ],
    )
    def k(ctx_hbm, ci_hbm, w_hbm, c_hbm, out_hbm,
          wi_v, ci_v, wrows, crows, out_v, sem):
        wid = lax.axis_index("s") * _NC + lax.axis_index("c")

        def chunk_body(ch, _):
            base = wid * RW + ch * NB
            pltpu.sync_copy(ctx_hbm.at[pl.ds(base * CTX, NB * CTX)], wi_v)
            pltpu.sync_copy(ci_hbm.at[pl.ds(base * NSCORE, NB * NSCORE)], ci_v)
            _gather_rows(w_hbm, wi_v, wrows, sem, NB * CTX)
            _gather_rows(c_hbm, ci_v, crows, sem, NB * NSCORE)

            def row_body(r, _):
                wb = r * CTX
                cb = r * NSCORE
                ctx_e = []
                for g in range(NG):
                    gs = pl.ds(g * _L, _L)
                    acc = wrows[wb, gs]
                    for j in range(1, CTX):
                        acc = acc + wrows[wb + j, gs]
                    ctx_e.append(acc * (1.0 / CTX))
                lane = lax.iota(jnp.int32, _L)
                svecs = [jnp.zeros((_L,), jnp.float32) for _ in range(OW // _L)]
                for n in range(NSCORE):
                    p = crows[cb + n, pl.ds(0, _L)] * ctx_e[0]
                    for g in range(1, NG):
                        p = p + crows[cb + n, pl.ds(g * _L, _L)] * ctx_e[g]
                    for sh in (8, 4, 2, 1):
                        p = p + jnp.take(p, lane ^ sh, mode="wrap")
                    v = n // _L
                    svecs[v] = jnp.where(lane == (n % _L), p, svecs[v])
                for v in range(OW // _L):
                    out_v[r, pl.ds(v * _L, _L)] = svecs[v]
                return 0

            lax.fori_loop(0, NB, row_body, 0)
            pltpu.sync_copy(out_v, out_hbm.at[pl.ds(base, NB)])
            return 0

        lax.fori_loop(0, NCHUNK, chunk_body, 0)

    return k(ctx_idx, ci, W, C)


def kernel(context, target, neg_samples, W, C):
    B, CTX = context.shape
    NEG = neg_samples.shape[1]
    ci = jnp.concatenate([target[:, None], neg_samples], axis=1).reshape(-1)
    logits = _cbow_logits(context.reshape(-1), ci, W, C, B, CTX, 1 + NEG)
    logits = logits[:, : 1 + NEG]
    labels = jnp.concatenate(
        [jnp.ones((B, 1), jnp.float32), jnp.zeros((B, NEG), jnp.float32)],
        axis=1,
    )
    return (logits, labels)


# speedup vs baseline: 5.2426x; 1.0375x over previous
import functools

import jax
import jax.numpy as jnp
import numpy as np
from jax import lax
from jax.experimental import pallas as pl
from jax.experimental.pallas import tpu as pltpu
from jax.experimental.pallas import tpu_sc as plsc

_NC = 2
_NS = 16
_L = 16


def _tree_sum(vals):
    vals = list(vals)
    while len(vals) > 1:
        nxt = [vals[i] + vals[i + 1] for i in range(0, len(vals) - 1, 2)]
        if len(vals) % 2:
            nxt.append(vals[-1])
        vals = nxt
    return vals[0]


def _gather_segments(total):
    segs = []
    off = 0
    while off < total:
        cnt = min(128, total - off)
        segs.append((off, cnt))
        off += cnt
    return segs


@functools.partial(jax.jit, static_argnums=(4, 5, 6))
def _cbow_logits(ctx_idx, ci, W, C, B, CTX, NSCORE):
    D = W.shape[1]
    NW = _NC * _NS
    RW = B // NW
    NB = 16
    NCHUNK = RW // NB
    NG = D // _L
    OW = 2 * _L

    mesh = plsc.VectorSubcoreMesh(
        core_axis_name="c", subcore_axis_name="s",
        num_cores=_NC, num_subcores=_NS,
    )

    @functools.partial(
        pl.kernel,
        out_type=jax.ShapeDtypeStruct((B, OW), jnp.float32),
        mesh=mesh,
        compiler_params=pltpu.CompilerParams(use_tc_tiling_on_sc=False),
        scratch_types=[
            pltpu.VMEM((RW * CTX,), jnp.int32),
            pltpu.VMEM((RW * NSCORE,), jnp.int32),
            pltpu.VMEM((NB * CTX, D), jnp.float32),
            pltpu.VMEM((NB * CTX, D), jnp.float32),
            pltpu.VMEM((NB * NSCORE, D), jnp.float32),
            pltpu.VMEM((NB * NSCORE, D), jnp.float32),
            pltpu.VMEM((NB, OW), jnp.float32),
            pltpu.SemaphoreType.DMA,
            pltpu.SemaphoreType.DMA,
        ],
    )
    def k(ctx_hbm, ci_hbm, w_hbm, c_hbm, out_hbm,
          wi_all, ci_all, wrows0, wrows1, crows0, crows1, out_v, sem0, sem1):
        wid = lax.axis_index("s") * _NC + lax.axis_index("c")
        wrows = (wrows0, wrows1)
        crows = (crows0, crows1)
        sems = (sem0, sem1)

        pltpu.sync_copy(ctx_hbm.at[pl.ds(wid * RW * CTX, RW * CTX)], wi_all)
        pltpu.sync_copy(ci_hbm.at[pl.ds(wid * RW * NSCORE, RW * NSCORE)],
                        ci_all)

        def make_copies(ch, slot):
            cps = []
            for off, cnt in _gather_segments(NB * CTX):
                cps.append(pltpu.make_async_copy(
                    w_hbm.at[wi_all.at[pl.ds(ch * NB * CTX + off, cnt)]],
                    wrows[slot].at[pl.ds(off, cnt)],
                    sems[slot]))
            for off, cnt in _gather_segments(NB * NSCORE):
                cps.append(pltpu.make_async_copy(
                    c_hbm.at[ci_all.at[pl.ds(ch * NB * NSCORE + off, cnt)]],
                    crows[slot].at[pl.ds(off, cnt)],
                    sems[slot]))
            return cps

        def issue(ch, slot):
            for cp in make_copies(ch, slot):
                cp.start()

        def drain(ch, slot):
            for cp in make_copies(ch, slot):
                cp.wait()

        def compute(ch, slot):
            wr = wrows[slot]
            cr = crows[slot]

            def row_body(r, _):
                lane = lax.iota(jnp.int32, _L)
                wb = r * CTX
                cb = r * NSCORE
                ctx_e = []
                for g in range(NG):
                    gs = pl.ds(g * _L, _L)
                    ctx_e.append(
                        _tree_sum([wr[wb + j, gs] for j in range(CTX)])
                        * (1.0 / CTX))
                masked = [[] for _ in range(OW // _L)]
                for n in range(NSCORE):
                    p = _tree_sum([
                        cr[cb + n, pl.ds(g * _L, _L)] * ctx_e[g]
                        for g in range(NG)])
                    for sh in (8, 4, 2, 1):
                        p = p + jnp.take(p, lane ^ sh, mode="wrap")
                    masked[n // _L].append(
                        jnp.where(lane == (n % _L), p, 0.0))
                for v in range(OW // _L):
                    out_v[r, pl.ds(v * _L, _L)] = _tree_sum(masked[v])
                return 0

            lax.fori_loop(0, NB, row_body, 0)
            base = wid * RW + ch * NB
            pltpu.sync_copy(out_v, out_hbm.at[pl.ds(base, NB)])

        def chunk_body(ch, _):
            issue(ch, 0)
            drain(ch, 0)
            compute(ch, 0)
            return 0

        lax.fori_loop(0, NCHUNK, chunk_body, 0)

    return k(ctx_idx, ci, W, C)


def kernel(context, target, neg_samples, W, C):
    B, CTX = context.shape
    NEG = neg_samples.shape[1]
    ci = jnp.concatenate([target[:, None], neg_samples], axis=1).reshape(-1)
    logits = _cbow_logits(context.reshape(-1), ci, W, C, B, CTX, 1 + NEG)
    logits = logits[:, : 1 + NEG]
    labels = jnp.concatenate(
        [jnp.ones((B, 1), jnp.float32), jnp.zeros((B, NEG), jnp.float32)],
        axis=1,
    )
    return (logits, labels)


# speedup vs baseline: 5.5365x; 1.0561x over previous
import functools

import jax
import jax.numpy as jnp
import numpy as np
from jax import lax
from jax.experimental import pallas as pl
from jax.experimental.pallas import tpu as pltpu
from jax.experimental.pallas import tpu_sc as plsc

_NC = 2
_NS = 16
_L = 16


def _tree_sum(vals):
    vals = list(vals)
    while len(vals) > 1:
        nxt = [vals[i] + vals[i + 1] for i in range(0, len(vals) - 1, 2)]
        if len(vals) % 2:
            nxt.append(vals[-1])
        vals = nxt
    return vals[0]


def _gather_segments(total):
    segs = []
    off = 0
    while off < total:
        cnt = min(128, total - off)
        segs.append((off, cnt))
        off += cnt
    return segs


@functools.partial(jax.jit, static_argnums=(4, 5, 6))
def _cbow_logits(ctx_idx, ci, W, C, B, CTX, NSCORE):
    D = W.shape[1]
    NW = _NC * _NS
    RW = B // NW
    NB = 16
    NCHUNK = RW // NB
    NG = D // _L
    OW = 2 * _L

    mesh = plsc.VectorSubcoreMesh(
        core_axis_name="c", subcore_axis_name="s",
        num_cores=_NC, num_subcores=_NS,
    )

    @functools.partial(
        pl.kernel,
        out_type=jax.ShapeDtypeStruct((B, OW), jnp.float32),
        mesh=mesh,
        compiler_params=pltpu.CompilerParams(use_tc_tiling_on_sc=False),
        scratch_types=[
            pltpu.VMEM((RW * CTX,), jnp.int32),
            pltpu.VMEM((RW * NSCORE,), jnp.int32),
            pltpu.VMEM((NB * CTX, D), jnp.float32),
            pltpu.VMEM((NB * CTX, D), jnp.float32),
            pltpu.VMEM((NB * NSCORE, D), jnp.float32),
            pltpu.VMEM((NB * NSCORE, D), jnp.float32),
            pltpu.VMEM((NB, OW), jnp.float32),
            pltpu.SemaphoreType.DMA,
            pltpu.SemaphoreType.DMA,
        ],
    )
    def k(ctx_hbm, ci_hbm, w_hbm, c_hbm, out_hbm,
          wi_all, ci_all, wrows0, wrows1, crows0, crows1, out_v, sem0, sem1):
        wid = lax.axis_index("s") * _NC + lax.axis_index("c")
        wrows = (wrows0, wrows1)
        crows = (crows0, crows1)
        sems = (sem0, sem1)

        pltpu.sync_copy(ctx_hbm.at[pl.ds(wid * RW * CTX, RW * CTX)], wi_all)
        pltpu.sync_copy(ci_hbm.at[pl.ds(wid * RW * NSCORE, RW * NSCORE)],
                        ci_all)

        def make_copies(ch, slot):
            cps = []
            for off, cnt in _gather_segments(NB * CTX):
                cps.append(pltpu.make_async_copy(
                    w_hbm.at[wi_all.at[pl.ds(ch * NB * CTX + off, cnt)]],
                    wrows[slot].at[pl.ds(off, cnt)],
                    sems[slot]))
            for off, cnt in _gather_segments(NB * NSCORE):
                cps.append(pltpu.make_async_copy(
                    c_hbm.at[ci_all.at[pl.ds(ch * NB * NSCORE + off, cnt)]],
                    crows[slot].at[pl.ds(off, cnt)],
                    sems[slot]))
            return cps

        def issue(ch, slot):
            for cp in make_copies(ch, slot):
                cp.start()

        def drain(ch, slot):
            for cp in make_copies(ch, slot):
                cp.wait()

        def compute(ch, slot):
            wr = wrows[slot]
            cr = crows[slot]

            def row_body(r, _):
                lane = lax.iota(jnp.int32, _L)
                wb = r * CTX
                cb = r * NSCORE
                ctx_e = []
                for g in range(NG):
                    gs = pl.ds(g * _L, _L)
                    ctx_e.append(
                        _tree_sum([wr[wb + j, gs] for j in range(CTX)])
                        * (1.0 / CTX))
                masked = [[] for _ in range(OW // _L)]
                for n in range(NSCORE):
                    p = _tree_sum([
                        cr[cb + n, pl.ds(g * _L, _L)] * ctx_e[g]
                        for g in range(NG)])
                    for sh in (8, 4, 2, 1):
                        p = p + jnp.take(p, lane ^ sh, mode="wrap")
                    masked[n // _L].append(
                        jnp.where(lane == (n % _L), p, 0.0))
                for v in range(OW // _L):
                    out_v[r, pl.ds(v * _L, _L)] = _tree_sum(masked[v])
                return 0

            lax.fori_loop(0, NB, row_body, 0)
            base = wid * RW + ch * NB
            pltpu.sync_copy(out_v, out_hbm.at[pl.ds(base, NB)])

        issue(0, 0)
        issue(1, 1)

        def pair_body(cp_i, _):
            for slot in (0, 1):
                ch = cp_i * 2 + slot
                drain(ch, slot)
                compute(ch, slot)

                @pl.when(ch + 2 < NCHUNK)
                def _():
                    issue(ch + 2, slot)
            return 0

        lax.fori_loop(0, NCHUNK // 2, pair_body, 0)

    return k(ctx_idx, ci, W, C)


def kernel(context, target, neg_samples, W, C):
    B, CTX = context.shape
    NEG = neg_samples.shape[1]
    ci = jnp.concatenate([target[:, None], neg_samples], axis=1).reshape(-1)
    logits = _cbow_logits(context.reshape(-1), ci, W, C, B, CTX, 1 + NEG)
    logits = logits[:, : 1 + NEG]
    labels = jnp.concatenate(
        [jnp.ones((B, 1), jnp.float32), jnp.zeros((B, NEG), jnp.float32)],
        axis=1,
    )
    return (logits, labels)
